# baseline (speedup 1.0000x reference)
import functools

import jax
import jax.numpy as jnp
from jax import lax
from jax.experimental import pallas as pl
from jax.experimental.pallas import tpu as pltpu
from jax.experimental.pallas import tpu_sc as plsc

VOCAB = 32
D = 256
NTOK = 64 * 2048
NC = 2
NS = 16
NW = NC * NS
B_PER_W = NTOK // NW
LANES = 16
SCALE = 16.0

_mesh = plsc.VectorSubcoreMesh(core_axis_name="c", subcore_axis_name="s")


@functools.partial(
    pl.kernel,
    out_type=jax.ShapeDtypeStruct((NTOK, D), jnp.float32),
    mesh=_mesh,
    scratch_types=dict(
        ltab=pltpu.VMEM((VOCAB, D), jnp.float32),
        idx_v=pltpu.VMEM((B_PER_W,), jnp.int32),
        s0=pltpu.SemaphoreType.DMA,
        s_idx=pltpu.SemaphoreType.DMA,
    ),
    compiler_params=pltpu.CompilerParams(needs_layout_passes=False),
)
def _emb_kernel(x_hbm, tabf_hbm, out_hbm, ltab, idx_v, s0, s_idx):
    c = lax.axis_index("c")
    s = lax.axis_index("s")
    wid = s * NC + c
    base = wid * B_PER_W

    idx_cp = pltpu.make_async_copy(
        x_hbm.at[pl.ds(base, B_PER_W)], idx_v, s_idx)
    idx_cp.start()
    pltpu.sync_copy(tabf_hbm, ltab)
    for r in range(VOCAB):
        for j in range(D // LANES):
            v = ltab[r, pl.ds(j * LANES, LANES)]
            ltab[r, pl.ds(j * LANES, LANES)] = v * SCALE
    idx_cp.wait()

    def group(gi, carry):
        rows = idx_v[pl.ds(gi * LANES, LANES)]
        for t in range(LANES):
            r = rows[t]
            pltpu.make_async_copy(
                ltab.at[r],
                out_hbm.at[base + gi * LANES + t],
                s0,
            ).start()
        return carry
    lax.fori_loop(0, B_PER_W // LANES, group, 0)

    pltpu.make_async_copy(
        out_hbm.at[pl.ds(0, B_PER_W)],
        out_hbm.at[pl.ds(base, B_PER_W)],
        s0,
    ).wait()


def kernel(x, table):
    b, sq = x.shape
    out = _emb_kernel(x.reshape(-1).astype(jnp.int32), table)
    return out.reshape(b, sq, D)


# speedup vs baseline: 1.0384x; 1.0384x over previous
import functools

import jax
import jax.numpy as jnp
from jax import lax
from jax.experimental import pallas as pl
from jax.experimental.pallas import tpu as pltpu
from jax.experimental.pallas import tpu_sc as plsc

VOCAB = 32
D = 256
NTOK = 64 * 2048
NC = 2
NS = 16
NW = NC * NS
B_PER_W = NTOK // NW
LANES = 16
SCALE = 16.0

_mesh = plsc.VectorSubcoreMesh(core_axis_name="c", subcore_axis_name="s")


@functools.partial(
    pl.kernel,
    out_type=jax.ShapeDtypeStruct((NTOK, D), jnp.float32),
    mesh=_mesh,
    scratch_types=dict(
        ltab=pltpu.VMEM((VOCAB, D), jnp.float32),
        idx_v=pltpu.VMEM((B_PER_W,), jnp.int32),
        s0=pltpu.SemaphoreType.DMA,
        s_idx=pltpu.SemaphoreType.DMA,
    ),
    compiler_params=pltpu.CompilerParams(needs_layout_passes=False),
)
def _emb_kernel(x_hbm, tabf_hbm, out_hbm, ltab, idx_v, s0, s_idx):
    c = lax.axis_index("c")
    s = lax.axis_index("s")
    wid = s * NC + c
    base = wid * B_PER_W

    idx_cp = pltpu.make_async_copy(
        x_hbm.at[pl.ds(base, B_PER_W)], idx_v, s_idx)
    idx_cp.start()
    pltpu.sync_copy(tabf_hbm, ltab)

    def scale_body(r, carry):
        def col_body(j, carry2):
            v = ltab[r, pl.ds(j * LANES, LANES)]
            ltab[r, pl.ds(j * LANES, LANES)] = v * SCALE
            return carry2
        return lax.fori_loop(0, D // LANES, col_body, carry)
    lax.fori_loop(0, VOCAB, scale_body, 0)
    idx_cp.wait()

    def group(gi, carry):
        rows = idx_v[pl.ds(gi * LANES, LANES)]
        for t in range(LANES):
            r = rows[t]
            pltpu.make_async_copy(
                ltab.at[r],
                out_hbm.at[base + gi * LANES + t],
                s0,
            ).start()
        return carry
    lax.fori_loop(0, B_PER_W // LANES, group, 0)

    pltpu.make_async_copy(
        out_hbm.at[pl.ds(0, B_PER_W)],
        out_hbm.at[pl.ds(base, B_PER_W)],
        s0,
    ).wait()


def kernel(x, table):
    b, sq = x.shape
    out = _emb_kernel(x.reshape(-1).astype(jnp.int32), table)
    return out.reshape(b, sq, D)
